# baseline (speedup 1.0000x reference)
import dataclasses
import functools

import jax
import jax.numpy as jnp
from jax import lax
from jax.experimental import pallas as pl
from jax.experimental.pallas import tpu as pltpu
from jax.experimental.pallas import tpu_sc as plsc

_L = 16
_COLS = 128
_FSEG = 25
_NSEG = 4
_NACC = _L + 1


@functools.partial(jax.jit, static_argnums=(1, 2, 3))
def _sc_pool_t(xt, b, f, d):
    mesh = plsc.VectorSubcoreMesh(core_axis_name="core", subcore_axis_name="subcore")
    cp = pltpu.CompilerParams()
    if "needs_layout_passes" in pltpu.CompilerParams.__dataclass_fields__:
        cp = dataclasses.replace(cp, needs_layout_passes=False)
    seg_rows = _FSEG * d
    n_lg = _COLS // _L

    @functools.partial(
        pl.kernel,
        out_type=jax.ShapeDtypeStruct((b,), jnp.float32),
        mesh=mesh,
        compiler_params=cp,
        scratch_types=[pltpu.VMEM((n_lg * _NACC * _L,), jnp.float32)],
    )
    def k(x_hbm, o_hbm, acc_ref):
        def body(x_vmem, o_vmem, acc):
            r = pl.program_id(1)
            first = r == 0

            @pl.loop(0, n_lg)
            def per_lane_group(g):
                base = g * (_NACC * _L)
                ss = [
                    jnp.where(first, 0.0, acc[pl.ds(base + t * _L, _L)])
                    for t in range(_L)
                ]
                qs = (
                    jnp.where(first, 0.0, acc[pl.ds(base + _L * _L, _L)]),
                    jnp.zeros((_L,), jnp.float32),
                    jnp.zeros((_L,), jnp.float32),
                    jnp.zeros((_L,), jnp.float32),
                )

                def fstep(fi, carry):
                    s = list(carry[:_L])
                    q = list(carry[_L:])
                    for fu in range(5):
                        row = (fi * 5 + fu) * d
                        for dd in range(d):
                            v = x_vmem[row + dd, pl.ds(g * _L, _L)]
                            s[dd] = s[dd] + v
                            q[dd % 4] = q[dd % 4] + v * v
                    return tuple(s) + tuple(q)

                state = lax.fori_loop(0, _FSEG // 5, fstep, tuple(ss) + qs)
                ss = state[:_L]
                q = (state[_L] + state[_L + 1]) + (state[_L + 2] + state[_L + 3])
                for t in range(_L):
                    acc[pl.ds(base + t * _L, _L)] = ss[t]
                acc[pl.ds(base + _L * _L, _L)] = q
                tot = ss[0] * ss[0]
                for t in range(1, _L):
                    tot = tot + ss[t] * ss[t]
                o_vmem[pl.ds(g * _L, _L)] = (tot - q) * 0.5

        pltpu.emit_pipeline(
            body,
            grid=(b // _COLS, _NSEG),
            in_specs=[pl.BlockSpec((seg_rows, _COLS), lambda i, j: (j, i))],
            out_specs=[pl.BlockSpec((_COLS,), lambda i, j: (i,))],
            core_axis_name=("core", "subcore"),
            dimension_semantics=(pltpu.PARALLEL, pltpu.ARBITRARY),
        )(x_hbm, o_hbm, scratches=[acc_ref])

    return k(xt)


def kernel(feature_emb):
    b, f, d = feature_emb.shape
    xt = feature_emb.transpose(1, 2, 0).reshape(f * d, b)
    return _sc_pool_t(xt, b, f, d).reshape(b, 1)


# speedup vs baseline: 1.7696x; 1.7696x over previous
import dataclasses
import functools

import jax
import jax.numpy as jnp
from jax import lax
from jax.experimental import pallas as pl
from jax.experimental.pallas import tpu as pltpu
from jax.experimental.pallas import tpu_sc as plsc

_L = 16
_COLS = 128
_FSEG = 25
_NSEG = 4
_NACC = _L + 1
_TC_COLS = 12288
_TC_BLK = 512


def _sc_pool_t(xt, b, f, d, col0, ncols):
    mesh = plsc.VectorSubcoreMesh(core_axis_name="core", subcore_axis_name="subcore")
    cp = pltpu.CompilerParams()
    if "needs_layout_passes" in pltpu.CompilerParams.__dataclass_fields__:
        cp = dataclasses.replace(cp, needs_layout_passes=False)
    seg_rows = _FSEG * d
    n_lg = _COLS // _L
    chunk0 = col0 // _COLS

    @functools.partial(
        pl.kernel,
        out_type=jax.ShapeDtypeStruct((ncols,), jnp.float32),
        mesh=mesh,
        compiler_params=cp,
        scratch_types=[pltpu.VMEM((n_lg * _NACC * _L,), jnp.float32)],
    )
    def k(x_hbm, o_hbm, acc_ref):
        def body(x_vmem, o_vmem, acc):
            r = pl.program_id(1)
            first = r == 0

            @pl.loop(0, n_lg)
            def per_lane_group(g):
                base = g * (_NACC * _L)
                ss = [
                    jnp.where(first, 0.0, acc[pl.ds(base + t * _L, _L)])
                    for t in range(_L)
                ]
                qs = (
                    jnp.where(first, 0.0, acc[pl.ds(base + _L * _L, _L)]),
                    jnp.zeros((_L,), jnp.float32),
                    jnp.zeros((_L,), jnp.float32),
                    jnp.zeros((_L,), jnp.float32),
                )

                def fstep(fi, carry):
                    row = fi * d
                    s = list(carry[:_L])
                    q = list(carry[_L:])
                    for dd in range(d):
                        v = x_vmem[row + dd, pl.ds(g * _L, _L)]
                        s[dd] = s[dd] + v
                        q[dd % 4] = q[dd % 4] + v * v
                    return tuple(s) + tuple(q)

                state = lax.fori_loop(0, _FSEG, fstep, tuple(ss) + qs)
                ss = state[:_L]
                q = (state[_L] + state[_L + 1]) + (state[_L + 2] + state[_L + 3])
                for t in range(_L):
                    acc[pl.ds(base + t * _L, _L)] = ss[t]
                acc[pl.ds(base + _L * _L, _L)] = q
                tot = ss[0] * ss[0]
                for t in range(1, _L):
                    tot = tot + ss[t] * ss[t]
                o_vmem[pl.ds(g * _L, _L)] = (tot - q) * 0.5

        pltpu.emit_pipeline(
            body,
            grid=(ncols // _COLS, _NSEG),
            in_specs=[pl.BlockSpec((seg_rows, _COLS), lambda i, j: (j, i + chunk0))],
            out_specs=[pl.BlockSpec((_COLS,), lambda i, j: (i,))],
            core_axis_name=("core", "subcore"),
            dimension_semantics=(pltpu.PARALLEL, pltpu.ARBITRARY),
        )(x_hbm, o_hbm, scratches=[acc_ref])

    return k(xt)


def _tc_pool_t(xt, b, f, d, ncols):
    def tck(x_ref, o_ref):
        xb = x_ref[...]
        x3 = xb.reshape(f, d, _TC_BLK)
        s = jnp.sum(x3, axis=0)
        q = jnp.sum(xb * xb, axis=0)
        o_ref[0, :] = (jnp.sum(s * s, axis=0) - q) * 0.5

    return pl.pallas_call(
        tck,
        grid=(ncols // _TC_BLK,),
        in_specs=[pl.BlockSpec((f * d, _TC_BLK), lambda j: (0, j))],
        out_specs=pl.BlockSpec((1, _TC_BLK), lambda j: (0, j)),
        out_shape=jax.ShapeDtypeStruct((1, ncols), jnp.float32),
    )(xt)


@functools.partial(jax.jit, static_argnums=(1, 2, 3))
def _pool(x, b, f, d):
    xt = x.transpose(1, 2, 0).reshape(f * d, b)
    sc_out = _sc_pool_t(xt, b, f, d, _TC_COLS, b - _TC_COLS)
    tc_out = _tc_pool_t(xt, b, f, d, _TC_COLS)
    return jnp.concatenate([tc_out.reshape(-1), sc_out]).reshape(b, 1)


def kernel(feature_emb):
    b, f, d = feature_emb.shape
    return _pool(feature_emb, b, f, d)


# speedup vs baseline: 1.7786x; 1.0051x over previous
import dataclasses
import functools

import jax
import jax.numpy as jnp
from jax import lax
from jax.experimental import pallas as pl
from jax.experimental.pallas import tpu as pltpu
from jax.experimental.pallas import tpu_sc as plsc

_L = 16
_COLS = 128
_FSEG = 25
_NSEG = 4
_NACC = _L + 1
_TC_COLS = 12288
_TC_BLK = 512


def _sc_pool_t(xt, b, f, d, col0, ncols):
    mesh = plsc.VectorSubcoreMesh(core_axis_name="core", subcore_axis_name="subcore")
    cp = pltpu.CompilerParams()
    if "needs_layout_passes" in pltpu.CompilerParams.__dataclass_fields__:
        cp = dataclasses.replace(cp, needs_layout_passes=False)
    seg_rows = _FSEG * d
    n_lg = _COLS // _L
    chunk0 = col0 // _COLS

    @functools.partial(
        pl.kernel,
        out_type=jax.ShapeDtypeStruct((ncols,), jnp.float32),
        mesh=mesh,
        compiler_params=cp,
        scratch_types=[pltpu.VMEM((n_lg * _NACC * _L,), jnp.float32)],
    )
    def k(x_hbm, o_hbm, acc_ref):
        def body(x_vmem, o_vmem, acc):
            r = pl.program_id(1)
            first = r == 0

            @pl.loop(0, n_lg)
            def per_lane_group(g):
                base = g * (_NACC * _L)
                ss = [
                    jnp.where(first, 0.0, acc[pl.ds(base + t * _L, _L)])
                    for t in range(_L)
                ]
                qs = (
                    jnp.where(first, 0.0, acc[pl.ds(base + _L * _L, _L)]),
                    jnp.zeros((_L,), jnp.float32),
                    jnp.zeros((_L,), jnp.float32),
                    jnp.zeros((_L,), jnp.float32),
                )

                def fstep(fi, carry):
                    row = fi * d
                    s = list(carry[:_L])
                    q = list(carry[_L:])
                    for dd in range(d):
                        v = x_vmem[row + dd, pl.ds(g * _L, _L)]
                        s[dd] = s[dd] + v
                        q[dd % 4] = q[dd % 4] + v * v
                    return tuple(s) + tuple(q)

                state = lax.fori_loop(0, _FSEG, fstep, tuple(ss) + qs)
                ss = state[:_L]
                q = (state[_L] + state[_L + 1]) + (state[_L + 2] + state[_L + 3])
                for t in range(_L):
                    acc[pl.ds(base + t * _L, _L)] = ss[t]
                acc[pl.ds(base + _L * _L, _L)] = q
                tot = ss[0] * ss[0]
                for t in range(1, _L):
                    tot = tot + ss[t] * ss[t]
                o_vmem[pl.ds(g * _L, _L)] = (tot - q) * 0.5

        pltpu.emit_pipeline(
            body,
            grid=(ncols // _COLS, _NSEG),
            in_specs=[pl.BlockSpec((seg_rows, _COLS), lambda i, j: (j, i + chunk0))],
            out_specs=[pl.BlockSpec((_COLS,), lambda i, j: (i,))],
            core_axis_name=("core", "subcore"),
            dimension_semantics=(pltpu.PARALLEL, pltpu.ARBITRARY),
        )(x_hbm, o_hbm, scratches=[acc_ref])

    return k(xt)


def _tc_pool_t(xt, b, f, d, ncols):
    def tck(x_ref, o_ref):
        s = x_ref[pl.ds(0, d), :]
        s2 = s * s
        for jf in range(1, f):
            v = x_ref[pl.ds(jf * d, d), :]
            s = s + v
            s2 = s2 + v * v
        o_ref[0, :] = (jnp.sum(s * s, axis=0) - jnp.sum(s2, axis=0)) * 0.5

    return pl.pallas_call(
        tck,
        grid=(ncols // _TC_BLK,),
        in_specs=[pl.BlockSpec((f * d, _TC_BLK), lambda j: (0, j))],
        out_specs=pl.BlockSpec((1, _TC_BLK), lambda j: (0, j)),
        out_shape=jax.ShapeDtypeStruct((1, ncols), jnp.float32),
    )(xt)


@functools.partial(jax.jit, static_argnums=(1, 2, 3))
def _pool(x, b, f, d):
    xt = x.transpose(1, 2, 0).reshape(f * d, b)
    sc_out = _sc_pool_t(xt, b, f, d, _TC_COLS, b - _TC_COLS)
    tc_out = _tc_pool_t(xt, b, f, d, _TC_COLS)
    return jnp.concatenate([tc_out.reshape(-1), sc_out]).reshape(b, 1)


def kernel(feature_emb):
    b, f, d = feature_emb.shape
    return _pool(feature_emb, b, f, d)


# speedup vs baseline: 2.0049x; 1.1272x over previous
import dataclasses
import functools

import jax
import jax.numpy as jnp
from jax import lax
from jax.experimental import pallas as pl
from jax.experimental.pallas import tpu as pltpu
from jax.experimental.pallas import tpu_sc as plsc

_L = 16
_COLS = 128
_FSEG = 25
_NSEG = 4
_NACC = _L + 1
_TC_COLS = 12288
_TC_BLK = 512


def _sc_pool_t(xt, b, f, d, col0, ncols):
    mesh = plsc.VectorSubcoreMesh(core_axis_name="core", subcore_axis_name="subcore")
    cp = pltpu.CompilerParams()
    if "needs_layout_passes" in pltpu.CompilerParams.__dataclass_fields__:
        cp = dataclasses.replace(cp, needs_layout_passes=False)
    seg_rows = _FSEG * d
    n_lg = _COLS // _L
    chunk0 = col0 // _COLS

    @functools.partial(
        pl.kernel,
        out_type=jax.ShapeDtypeStruct((ncols,), jnp.float32),
        mesh=mesh,
        compiler_params=cp,
        scratch_types=[pltpu.VMEM((n_lg * _NACC * _L,), jnp.float32)],
    )
    def k(x_hbm, o_hbm, acc_ref):
        def body(x_vmem, o_vmem, acc):
            r = pl.program_id(1)
            first = r == 0

            @pl.loop(0, n_lg)
            def per_lane_group(g):
                base = g * (_NACC * _L)
                ss = [
                    jnp.where(first, 0.0, acc[pl.ds(base + t * _L, _L)])
                    for t in range(_L)
                ]
                qs = (
                    jnp.where(first, 0.0, acc[pl.ds(base + _L * _L, _L)]),
                    jnp.zeros((_L,), jnp.float32),
                    jnp.zeros((_L,), jnp.float32),
                    jnp.zeros((_L,), jnp.float32),
                )

                def fstep(fi, carry):
                    row = fi * d
                    s = list(carry[:_L])
                    q = list(carry[_L:])
                    for dd in range(d):
                        v = x_vmem[row + dd, pl.ds(g * _L, _L)]
                        s[dd] = s[dd] + v
                        q[dd % 4] = q[dd % 4] + v * v
                    return tuple(s) + tuple(q)

                state = lax.fori_loop(0, _FSEG, fstep, tuple(ss) + qs)
                ss = state[:_L]
                q = (state[_L] + state[_L + 1]) + (state[_L + 2] + state[_L + 3])
                for t in range(_L):
                    acc[pl.ds(base + t * _L, _L)] = ss[t]
                acc[pl.ds(base + _L * _L, _L)] = q
                tot = ss[0] * ss[0]
                for t in range(1, _L):
                    tot = tot + ss[t] * ss[t]
                o_vmem[pl.ds(g * _L, _L)] = (tot - q) * 0.5

        pltpu.emit_pipeline(
            body,
            grid=(ncols // _COLS, _NSEG),
            in_specs=[pl.BlockSpec((seg_rows, _COLS), lambda i, j: (j, i + chunk0))],
            out_specs=[pl.BlockSpec((_COLS,), lambda i, j: (i,))],
            core_axis_name=("core", "subcore"),
            dimension_semantics=(pltpu.PARALLEL, pltpu.ARBITRARY),
        )(x_hbm, o_hbm, scratches=[acc_ref])

    return k(xt)


def _tc_pool_t(xt, b, f, d, ncols):
    fb = 10
    steps = f // fb

    def tck(x_ref, o_ref, s_ref, q_ref):
        r = pl.program_id(0)
        sl = x_ref[pl.ds(0, d), :]
        s2l = sl * sl
        for jf in range(1, fb):
            v = x_ref[pl.ds(jf * d, d), :]
            sl = sl + v
            s2l = s2l + v * v
        s = jnp.where(r == 0, sl, s_ref[...] + sl)
        q = jnp.where(r == 0, s2l, q_ref[...] + s2l)
        s_ref[...] = s
        q_ref[...] = q

        @pl.when(r == steps - 1)
        def _():
            o_ref[0, :] = (jnp.sum(s * s, axis=0) - jnp.sum(q, axis=0)) * 0.5

    return pl.pallas_call(
        tck,
        grid=(steps,),
        in_specs=[pl.BlockSpec((fb * d, ncols), lambda r: (r, 0))],
        out_specs=pl.BlockSpec((1, ncols), lambda r: (0, 0)),
        out_shape=jax.ShapeDtypeStruct((1, ncols), jnp.float32),
        scratch_shapes=[
            pltpu.VMEM((d, ncols), jnp.float32),
            pltpu.VMEM((d, ncols), jnp.float32),
        ],
    )(xt)


@functools.partial(jax.jit, static_argnums=(1, 2, 3))
def _pool(x, b, f, d):
    xt = x.transpose(1, 2, 0).reshape(f * d, b)
    sc_out = _sc_pool_t(xt, b, f, d, _TC_COLS, b - _TC_COLS)
    tc_out = _tc_pool_t(xt, b, f, d, _TC_COLS)
    return jnp.concatenate([tc_out.reshape(-1), sc_out]).reshape(b, 1)


def kernel(feature_emb):
    b, f, d = feature_emb.shape
    return _pool(feature_emb, b, f, d)
